# baseline (speedup 1.0000x reference)
import functools

import jax
import jax.numpy as jnp
from jax.experimental import pallas as pl
from jax.experimental.pallas import tpu as pltpu

_NEG = -1e30


def _softmax_rows(logits):
    p = jnp.exp(logits - jnp.max(logits, axis=1, keepdims=True))
    p = p / jnp.sum(p, axis=1, keepdims=True)
    return jnp.sum(p.reshape(-1, 8, p.shape[1]), axis=0)


def _cls_body(x_ref, w1_ref, w2_ref, b1_ref, b2_ref, n_ref, xb_ref,
              acc_ref, lg_ref, lga_ref):
    m, k = pl.program_id(0), pl.program_id(1)
    nk = pl.num_programs(1)
    rows = lg_ref.shape[0] // nk

    psum = _softmax_rows(lg_ref[pl.ds(k * rows, rows), :])
    acc_ref[...] = jnp.where(m == 0, jnp.zeros_like(psum),
                             acc_ref[...] + psum)

    @pl.when(k == 0)
    def _():
        xb_ref[...] = x_ref[...].astype(jnp.bfloat16)

    w1 = w1_ref[...].astype(jnp.bfloat16)
    h = (jnp.dot(xb_ref[...], w1, preferred_element_type=jnp.float32)
         + b1_ref[...])
    hb = jnp.maximum(h, 0.0).astype(jnp.bfloat16)
    part = jnp.dot(hb, w2_ref[...], preferred_element_type=jnp.float32)
    lga_ref[...] = jnp.where(k == 0, part, lga_ref[...] + part)

    @pl.when(k == nk - 1)
    def _():
        lg_ref[...] = lga_ref[...] + b2_ref[...]

    @pl.when(jnp.logical_and(m == pl.num_programs(0) - 1, k == nk - 1))
    def _():
        s = jnp.sum(acc_ref[...] + _softmax_rows(lga_ref[...] + b2_ref[...]),
                    axis=0)
        n_ref[0] = jnp.argmax(s).astype(jnp.int32)


def _expert_body(n_sref, x_ref, w1_ref, b1_ref, w2_ref, b2_ref, y_ref, c_ref):
    k = pl.program_id(1)

    y_ref[...] = jnp.where(k == 0, b2_ref[0].astype(jnp.float32),
                           y_ref[...] + c_ref[...])

    w1 = w1_ref[0].astype(jnp.bfloat16)
    hm = jnp.maximum(
        jnp.dot(x_ref[...], w1, preferred_element_type=jnp.float32) + b1_ref[0],
        0.0).astype(jnp.bfloat16)
    w2 = w2_ref[0].astype(jnp.bfloat16)
    contrib = jnp.dot(hm, w2, preferred_element_type=jnp.float32)
    c_ref[...] = contrib

    @pl.when(k == pl.num_programs(1) - 1)
    def _():
        y_ref[...] += contrib


@functools.partial(jax.jit, static_argnames=("interpret",))
def kernel(x, Wc1, bc1, Wc2, bc2, W1, b1, W2, b2, interpret=False):
    B, T, D = x.shape
    H = Wc1.shape[1]
    E, V = W2.shape[0], W2.shape[2]
    M = B * T
    xm = x.reshape(M, D)

    EP = 128
    Wc2p = jnp.zeros((H, EP), jnp.bfloat16).at[:, :E].set(Wc2.astype(jnp.bfloat16))
    bc2p = jnp.full((1, EP), _NEG, jnp.float32).at[0, :E].set(bc2)

    TM1 = 1024 if M % 1024 == 0 else M
    TH1 = 1024 if H % 1024 == 0 else H
    n_out, xb = pl.pallas_call(
        _cls_body,
        grid=(M // TM1, H // TH1),
        in_specs=[
            pl.BlockSpec((TM1, D), lambda m, k: (m, 0)),
            pl.BlockSpec((D, TH1), lambda m, k: (0, k)),
            pl.BlockSpec((TH1, EP), lambda m, k: (k, 0)),
            pl.BlockSpec((1, TH1), lambda m, k: (0, k)),
            pl.BlockSpec((1, EP), lambda m, k: (0, 0)),
        ],
        out_specs=[pl.BlockSpec(memory_space=pltpu.SMEM),
                   pl.BlockSpec((TM1, D), lambda m, k: (m, 0))],
        out_shape=[jax.ShapeDtypeStruct((1,), jnp.int32),
                   jax.ShapeDtypeStruct((M, D), jnp.bfloat16)],
        scratch_shapes=[pltpu.VMEM((8, EP), jnp.float32),
                        pltpu.VMEM((TM1, EP), jnp.float32),
                        pltpu.VMEM((TM1, EP), jnp.float32)],
        interpret=interpret,
    )(xm, Wc1, Wc2p, bc1.reshape(1, H), bc2p)

    n_arr = n_out

    TM2 = 2048 if M % 2048 == 0 else M
    TH = 1024 if H % 1024 == 0 else H
    grid_spec = pltpu.PrefetchScalarGridSpec(
        num_scalar_prefetch=1,
        grid=(M // TM2, H // TH),
        in_specs=[
            pl.BlockSpec((TM2, D), lambda m, k, n: (m, 0)),
            pl.BlockSpec((1, D, TH), lambda m, k, n: (n[0], 0, k)),
            pl.BlockSpec((1, 1, TH), lambda m, k, n: (n[0], 0, k)),
            pl.BlockSpec((1, TH, V), lambda m, k, n: (n[0], k, 0)),
            pl.BlockSpec((1, 1, V), lambda m, k, n: (n[0], 0, 0)),
        ],
        out_specs=pl.BlockSpec((TM2, V), lambda m, k, n: (m, 0)),
        scratch_shapes=[pltpu.VMEM((TM2, V), jnp.float32)],
    )
    y = pl.pallas_call(
        _expert_body,
        grid_spec=grid_spec,
        out_shape=jax.ShapeDtypeStruct((M, V), jnp.float32),
        interpret=interpret,
    )(n_arr, xb, W1, b1.reshape(E, 1, H), W2, b2.reshape(E, 1, V))

    return y.reshape(B, T, V)


# speedup vs baseline: 1.0312x; 1.0312x over previous
import functools

import jax
import jax.numpy as jnp
from jax.experimental import pallas as pl
from jax.experimental.pallas import tpu as pltpu

_NEG = -1e30


def _softmax_rows(logits):
    p = jnp.exp(logits - jnp.max(logits, axis=1, keepdims=True))
    p = p / jnp.sum(p, axis=1, keepdims=True)
    return jnp.sum(p.reshape(-1, 8, p.shape[1]), axis=0)


def _cls_body(x_ref, w1_ref, w2_ref, b1_ref, b2_ref, n_ref, xb_ref,
              acc_ref, lg_ref):
    m = pl.program_id(0)

    psum = _softmax_rows(lg_ref[...])
    acc_ref[...] = jnp.where(m == 0, jnp.zeros_like(psum),
                             acc_ref[...] + psum)

    xb = x_ref[...].astype(jnp.bfloat16)
    xb_ref[...] = xb
    h = (jnp.dot(xb, w1_ref[...], preferred_element_type=jnp.float32)
         + b1_ref[...])
    hb = jnp.maximum(h, 0.0).astype(jnp.bfloat16)
    logits = (jnp.dot(hb, w2_ref[...], preferred_element_type=jnp.float32)
              + b2_ref[...])
    lg_ref[...] = logits

    @pl.when(m == pl.num_programs(0) - 1)
    def _():
        s = jnp.sum(acc_ref[...] + _softmax_rows(logits), axis=0)
        n_ref[0] = jnp.argmax(s).astype(jnp.int32)


def _expert_body(n_sref, x_ref, w1_ref, b1_ref, w2_ref, b2_ref, y_ref, c_ref):
    k = pl.program_id(1)

    y_ref[...] = jnp.where(k == 0, b2_ref[0].astype(jnp.float32),
                           y_ref[...] + c_ref[...])

    w1 = w1_ref[0].astype(jnp.bfloat16)
    hm = jnp.maximum(
        jnp.dot(x_ref[...], w1, preferred_element_type=jnp.float32) + b1_ref[0],
        0.0).astype(jnp.bfloat16)
    w2 = w2_ref[0].astype(jnp.bfloat16)
    contrib = jnp.dot(hm, w2, preferred_element_type=jnp.float32)
    c_ref[...] = contrib

    @pl.when(k == pl.num_programs(1) - 1)
    def _():
        y_ref[...] += contrib


@functools.partial(jax.jit, static_argnames=("interpret",))
def kernel(x, Wc1, bc1, Wc2, bc2, W1, b1, W2, b2, interpret=False):
    B, T, D = x.shape
    H = Wc1.shape[1]
    E, V = W2.shape[0], W2.shape[2]
    M = B * T
    xm = x.reshape(M, D)

    EP = 128
    Wc2p = jnp.zeros((H, EP), jnp.bfloat16).at[:, :E].set(Wc2.astype(jnp.bfloat16))
    bc2p = jnp.full((1, EP), _NEG, jnp.float32).at[0, :E].set(bc2)

    Wc1b = Wc1.astype(jnp.bfloat16)
    TM1 = 1024 if M % 1024 == 0 else M
    n_out, xb = pl.pallas_call(
        _cls_body,
        grid=(M // TM1,),
        in_specs=[
            pl.BlockSpec((TM1, D), lambda m: (m, 0)),
            pl.BlockSpec((D, H), lambda m: (0, 0)),
            pl.BlockSpec((H, EP), lambda m: (0, 0)),
            pl.BlockSpec((1, H), lambda m: (0, 0)),
            pl.BlockSpec((1, EP), lambda m: (0, 0)),
        ],
        out_specs=[pl.BlockSpec(memory_space=pltpu.SMEM),
                   pl.BlockSpec((TM1, D), lambda m: (m, 0))],
        out_shape=[jax.ShapeDtypeStruct((1,), jnp.int32),
                   jax.ShapeDtypeStruct((M, D), jnp.bfloat16)],
        scratch_shapes=[pltpu.VMEM((8, EP), jnp.float32),
                        pltpu.VMEM((TM1, EP), jnp.float32)],
        interpret=interpret,
    )(xm, Wc1b, Wc2p, bc1.reshape(1, H), bc2p)

    n_arr = n_out

    TM2 = 2048 if M % 2048 == 0 else M
    TH = 1024 if H % 1024 == 0 else H
    grid_spec = pltpu.PrefetchScalarGridSpec(
        num_scalar_prefetch=1,
        grid=(M // TM2, H // TH),
        in_specs=[
            pl.BlockSpec((TM2, D), lambda m, k, n: (m, 0)),
            pl.BlockSpec((1, D, TH), lambda m, k, n: (n[0], 0, k)),
            pl.BlockSpec((1, 1, TH), lambda m, k, n: (n[0], 0, k)),
            pl.BlockSpec((1, TH, V), lambda m, k, n: (n[0], k, 0)),
            pl.BlockSpec((1, 1, V), lambda m, k, n: (n[0], 0, 0)),
        ],
        out_specs=pl.BlockSpec((TM2, V), lambda m, k, n: (m, 0)),
        scratch_shapes=[pltpu.VMEM((TM2, V), jnp.float32)],
    )
    y = pl.pallas_call(
        _expert_body,
        grid_spec=grid_spec,
        out_shape=jax.ShapeDtypeStruct((M, V), jnp.float32),
        interpret=interpret,
    )(n_arr, xb, W1, b1.reshape(E, 1, H), W2, b2.reshape(E, 1, V))

    return y.reshape(B, T, V)
